# baseline (speedup 1.0000x reference)
import functools

import jax
import jax.numpy as jnp
from jax import lax
from jax.experimental import pallas as pl
from jax.experimental.pallas import tpu as pltpu
from jax.experimental.pallas import tpu_sc as plsc

U = 256
N = 1024
P = 256
C = 256
D = 128
OUTD = 128
NC, NS, L = 2, 16, 16
NW = NC * NS
CHH = C // NC
BPS = U // NS
BPW = U // NW
NROW_S = N + 8
NROW_A = P + 8
I16MIN = -32768

_mesh = plsc.VectorSubcoreMesh(core_axis_name="c", subcore_axis_name="s")


@functools.partial(
    pl.kernel,
    out_type=(jax.ShapeDtypeStruct((U, CHH // 2), jnp.int32),
              jax.ShapeDtypeStruct((U, CHH // 2), jnp.int32),
              jax.ShapeDtypeStruct((U, D // 2), jnp.int32)),
    mesh=_mesh,
    scratch_types=[
        pltpu.VMEM((6 * U,), jnp.float32),
        pltpu.VMEM((N,), jnp.float32),
        pltpu.VMEM((N,), jnp.float32),
        pltpu.VMEM((N,), jnp.float32),
        pltpu.VMEM((P,), jnp.float32),
        pltpu.VMEM((P,), jnp.float32),
        pltpu.VMEM((P,), jnp.float32),
        pltpu.VMEM((NROW_S * CHH // 2,), jnp.int32),
        pltpu.VMEM((NROW_A * D // 2,), jnp.int32),
        pltpu.VMEM((BPS * N,), jnp.int32),
        pltpu.VMEM((BPW * P,), jnp.int32),
        pltpu.VMEM((BPS, CHH // 2), jnp.int32),
        pltpu.VMEM((BPW, D // 2), jnp.int32),
        pltpu.SemaphoreType.DMA,
        pltpu.SemaphoreType.DMA,
    ],
    compiler_params=pltpu.CompilerParams(needs_layout_passes=False),
)
def _sc_pool(ub_hbm, sx_hbm, sy_hbm, sz_hbm, ax_hbm, ay_hbm, az_hbm,
             sfk0_hbm, sfk1_hbm, bfk_hbm, g1a_hbm, g1b_hbm, g2_hbm,
             ub_v, sx_v, sy_v, sz_v, ax_v, ay_v, az_v,
             sfk_v, bfk_v, idxs_v, idxa_v, g1_v, g2_v, sem_s, sem_a):
    h = lax.axis_index("c")
    g = lax.axis_index("s")
    wid = g * NC + h
    ub_s = pl.multiple_of(g * BPS, BPS)
    ub_a = pl.multiple_of(wid * BPW, BPW)

    @pl.when(h == 0)
    def _():
        pltpu.async_copy(sfk0_hbm, sfk_v, sem_s)

    @pl.when(h != 0)
    def _():
        pltpu.async_copy(sfk1_hbm, sfk_v, sem_s)

    cp_s = pltpu.make_async_copy(sfk0_hbm, sfk_v, sem_s)
    cp_a = pltpu.async_copy(bfk_hbm, bfk_v, sem_a)
    pltpu.sync_copy(ub_hbm, ub_v)
    pltpu.sync_copy(sx_hbm, sx_v)
    pltpu.sync_copy(sy_hbm, sy_v)
    pltpu.sync_copy(sz_hbm, sz_v)
    pltpu.sync_copy(ax_hbm, ax_v)
    pltpu.sync_copy(ay_hbm, ay_v)
    pltpu.sync_copy(az_hbm, az_v)

    def compress(u, base, npts, xr, yr, zr, idx_ref):
        def bcast(r):
            return plsc.load_gather(
                ub_v, [jnp.full((L,), r * U + u, jnp.int32)])
        cx, cy, cz = bcast(0), bcast(1), bcast(2)
        hx, hy, hz = bcast(3) * 0.5, bcast(4) * 0.5, bcast(5) * 0.5
        lox, hix = cx - hx, cx + hx
        loy, hiy = cy - hy, cy + hy
        loz, hiz = cz - hz, cz + hz

        def mk(j, cnt):
            xv = xr[pl.ds(j * L, L)]
            yv = yr[pl.ds(j * L, L)]
            zv = zr[pl.ds(j * L, L)]
            m = ((xv >= lox) & (xv <= hix) & (yv >= loy) & (yv <= hiy)
                 & (zv >= loz) & (zv <= hiz))
            mi = m.astype(jnp.int32)
            cs = plsc.cumsum(mi)
            pos = (base + cnt + cs) - mi
            ids = lax.iota(jnp.int32, L) + j * L
            plsc.store_scatter(idx_ref, [pos], ids, mask=m)
            pc = plsc.all_reduce_population_count(m)
            return cnt + pc[0]
        cnt = lax.fori_loop(0, npts // L, mk, jnp.int32(0))

        pos = cnt + lax.iota(jnp.int32, L)
        plsc.store_scatter(idx_ref, [base + pos],
                           jnp.full((L,), npts, jnp.int32), mask=pos < npts)
        return cnt

    def pool(cnt, base, npts, idx_ref, tab_ref, nchan, out_ref, ob):
        nw = nchan // 2
        nvec = nw // L
        accs = tuple(jnp.full((2 * L,), I16MIN, jnp.int16)
                     for _ in range(nvec))

        def grp(t, accs):
            iv = idx_ref[pl.ds(base + t * L, L)]
            for lane in range(L):
                rb = iv[lane] * nw
                accs = tuple(
                    jnp.maximum(
                        accs[j],
                        plsc.bitcast(
                            tab_ref[pl.ds(
                                pl.multiple_of(rb + j * L, L), L)],
                            jnp.int16))
                    for j in range(nvec))
            return accs
        ngrp = (cnt + (L - 1)) >> 4
        accs = lax.fori_loop(0, ngrp, grp, accs)

        both_min = jnp.int32(-2147450880)
        fixw = jnp.full((L,), (cnt == npts).astype(jnp.int32) * both_min,
                        jnp.int32)
        fix = plsc.bitcast(fixw, jnp.int16)
        for j in range(nvec):
            out_ref[ob, pl.ds(j * L, L)] = plsc.bitcast(
                jnp.maximum(accs[j], fix), jnp.int32)

    cnts_s = [compress(ub_s + b, b * N, N, sx_v, sy_v, sz_v, idxs_v)
              for b in range(BPS)]
    cnts_a = [compress(ub_a + b, b * P, P, ax_v, ay_v, az_v, idxa_v)
              for b in range(BPW)]

    cp_a.wait()
    for b in range(BPW):
        pool(cnts_a[b], b * P, P, idxa_v, bfk_v, D, g2_v, b)
    cp_s.wait()
    for b in range(BPS):
        pool(cnts_s[b], b * N, N, idxs_v, sfk_v, CHH, g1_v, b)

    @pl.when(h == 0)
    def _():
        pltpu.sync_copy(g1_v, g1a_hbm.at[pl.ds(ub_s, BPS)])

    @pl.when(h != 0)
    def _():
        pltpu.sync_copy(g1_v, g1b_hbm.at[pl.ds(ub_s, BPS)])

    pltpu.sync_copy(g2_v, g2_hbm.at[pl.ds(ub_a, BPW)])


def _f16_key(x):
    u = lax.bitcast_convert_type(x, jnp.int32)
    s = lax.shift_right_logical(u, 31)
    e = lax.shift_right_logical(u, 23) & 0xFF
    m = u & 0x7FFFFF
    he = e - 112
    base = lax.shift_left(he, 10) | lax.shift_right_logical(m, 13)
    r = m & 0x1FFF
    lsb = lax.shift_right_logical(m, 13) & 1
    base = base + ((r > 0x1000) | ((r == 0x1000) & (lsb == 1))).astype(
        jnp.int32)
    base = jnp.where(he >= 31, 0x7C00, base)
    mf = m | 0x800000
    sh = jnp.clip(14 - he, 1, 31)
    subv = lax.shift_right_logical(mf, sh)
    subv = subv + (lax.shift_right_logical(mf, sh - 1) & 1)
    base = jnp.where(he <= 0, subv, base)
    return base ^ (0 - s)


def _key_to_f32(k):
    bits = jnp.where(k >= 0, k, 0x7FFF - k)
    s = lax.shift_right_logical(bits, 15) & 1
    e = lax.shift_right_logical(bits, 10) & 0x1F
    m = bits & 0x3FF
    fb = lax.shift_left(e + 112, 23) | lax.shift_left(m, 13)
    vn = lax.bitcast_convert_type(fb, jnp.float32)
    vs = m.astype(jnp.float32) * jnp.float32(2.0 ** -24)
    v = jnp.where(e == 0, vs, vn)
    return jnp.where(s == 1, -v, v)


def _pack_half(keys):
    lo = keys[:, 0:64] & 0xFFFF
    hi = lax.shift_left(keys[:, 64:128], 16)
    return lo | hi


_MINW = -2147450880


def _prep_body(sf_ref, bf_ref, sxyz_ref, axyz_ref, ub_ref,
               sfk0_ref, sfk1_ref, bfk_ref,
               sx_ref, sy_ref, sz_ref, ax_ref, ay_ref, az_ref, ub6_ref):
    skeys = _f16_key(sf_ref[:].T)
    sfk0_ref[0:N, :] = _pack_half(skeys[:, 0:CHH])
    sfk0_ref[N:NROW_S, :] = jnp.full((NROW_S - N, CHH // 2), _MINW, jnp.int32)
    sfk1_ref[0:N, :] = _pack_half(skeys[:, CHH:C])
    sfk1_ref[N:NROW_S, :] = jnp.full((NROW_S - N, CHH // 2), _MINW, jnp.int32)
    bkeys = _f16_key(bf_ref[:])
    bfk_ref[0:P, :] = _pack_half(bkeys)
    bfk_ref[P:NROW_A, :] = jnp.full((NROW_A - P, D // 2), _MINW, jnp.int32)
    st = sxyz_ref[:].T
    sx_ref[:], sy_ref[:], sz_ref[:] = st[0:1], st[1:2], st[2:3]
    at = axyz_ref[:].T
    ax_ref[:], ay_ref[:], az_ref[:] = at[0:1], at[1:2], at[2:3]
    ub6_ref[:] = ub_ref[:].T


def _finish_body(g1a_ref, g1b_ref, g2_ref, bfu_ref, w_ref, b_ref, out_ref):
    def unpack(w):
        lo = lax.shift_right_arithmetic(lax.shift_left(w, 16), 16)
        hi = lax.shift_right_arithmetic(w, 16)
        return _key_to_f32(lo), _key_to_f32(hi)

    a_lo, a_hi = unpack(g1a_ref[:])
    b_lo, b_hi = unpack(g1b_ref[:])
    c_lo, c_hi = unpack(g2_ref[:])
    glob = jnp.concatenate(
        [a_lo, a_hi, b_lo, b_hi, c_lo, c_hi, bfu_ref[:]], axis=1)
    out = lax.dot_general(glob, w_ref[:], (((1,), (1,)), ((), ())),
                          preferred_element_type=jnp.float32)
    out = out + b_ref[:]
    out_ref[:] = jax.nn.sigmoid(jnp.log(jnp.abs(out + 1e-6)))


def kernel(union_box, box_features, agg_xyz, seed_feature, seed_xyz,
           box_feature_union, W, b):
    f32, i32 = jnp.float32, jnp.int32
    sd = jax.ShapeDtypeStruct
    (sfk0, sfk1, bfk, sx, sy, sz, ax, ay, az, ub6) = pl.pallas_call(
        _prep_body,
        out_shape=(
            sd((NROW_S, CHH // 2), i32), sd((NROW_S, CHH // 2), i32),
            sd((NROW_A, D // 2), i32),
            sd((1, N), f32), sd((1, N), f32), sd((1, N), f32),
            sd((1, P), f32), sd((1, P), f32), sd((1, P), f32),
            sd((6, U), f32),
        ),
    )(seed_feature, box_features, seed_xyz, agg_xyz, union_box[0])

    g1a, g1b, g2k = _sc_pool(
        ub6.reshape(-1),
        sx.reshape(-1), sy.reshape(-1), sz.reshape(-1),
        ax.reshape(-1), ay.reshape(-1), az.reshape(-1),
        sfk0.reshape(-1), sfk1.reshape(-1), bfk.reshape(-1))

    return pl.pallas_call(
        _finish_body,
        out_shape=sd((U, OUTD), f32),
    )(g1a, g1b, g2k, box_feature_union[:, 0, :], W, b.reshape(1, OUTD))


# speedup vs baseline: 1.0529x; 1.0529x over previous
import functools

import jax
import jax.numpy as jnp
from jax import lax
from jax.experimental import pallas as pl
from jax.experimental.pallas import tpu as pltpu
from jax.experimental.pallas import tpu_sc as plsc

U = 256
N = 1024
P = 256
C = 256
D = 128
OUTD = 128
NC, NS, L = 2, 16, 16
NW = NC * NS
CHH = C // NC
BPS = U // NS
BPW = U // NW
NROW_S = N + 8
NROW_A = P + 8
I16MIN = -32768

_mesh = plsc.VectorSubcoreMesh(core_axis_name="c", subcore_axis_name="s")


@functools.partial(
    pl.kernel,
    out_type=(jax.ShapeDtypeStruct((U, CHH // 2), jnp.int32),
              jax.ShapeDtypeStruct((U, CHH // 2), jnp.int32),
              jax.ShapeDtypeStruct((U, D // 2), jnp.int32)),
    mesh=_mesh,
    scratch_types=[
        pltpu.VMEM((6 * U,), jnp.float32),
        pltpu.VMEM((N,), jnp.float32),
        pltpu.VMEM((N,), jnp.float32),
        pltpu.VMEM((N,), jnp.float32),
        pltpu.VMEM((P,), jnp.float32),
        pltpu.VMEM((P,), jnp.float32),
        pltpu.VMEM((P,), jnp.float32),
        pltpu.VMEM((NROW_S * CHH // 2,), jnp.int32),
        pltpu.VMEM((NROW_A * D // 2,), jnp.int32),
        pltpu.VMEM((BPS * N + L,), jnp.int32),
        pltpu.VMEM((BPW * P + L,), jnp.int32),
        pltpu.VMEM((BPS, CHH // 2), jnp.int32),
        pltpu.VMEM((BPW, D // 2), jnp.int32),
        pltpu.SemaphoreType.DMA,
        pltpu.SemaphoreType.DMA,
    ],
    compiler_params=pltpu.CompilerParams(needs_layout_passes=False),
)
def _sc_pool(ub_hbm, sx_hbm, sy_hbm, sz_hbm, ax_hbm, ay_hbm, az_hbm,
             sfk0_hbm, sfk1_hbm, bfk_hbm, g1a_hbm, g1b_hbm, g2_hbm,
             ub_v, sx_v, sy_v, sz_v, ax_v, ay_v, az_v,
             sfk_v, bfk_v, idxs_v, idxa_v, g1_v, g2_v, sem_s, sem_a):
    h = lax.axis_index("c")
    g = lax.axis_index("s")
    wid = g * NC + h
    ub_s = pl.multiple_of(g * BPS, BPS)
    ub_a = pl.multiple_of(wid * BPW, BPW)

    @pl.when(h == 0)
    def _():
        pltpu.async_copy(sfk0_hbm, sfk_v, sem_s)

    @pl.when(h != 0)
    def _():
        pltpu.async_copy(sfk1_hbm, sfk_v, sem_s)

    cp_s = pltpu.make_async_copy(sfk0_hbm, sfk_v, sem_s)
    cp_a = pltpu.async_copy(bfk_hbm, bfk_v, sem_a)
    pltpu.sync_copy(ub_hbm, ub_v)
    pltpu.sync_copy(sx_hbm, sx_v)
    pltpu.sync_copy(sy_hbm, sy_v)
    pltpu.sync_copy(sz_hbm, sz_v)
    pltpu.sync_copy(ax_hbm, ax_v)
    pltpu.sync_copy(ay_hbm, ay_v)
    pltpu.sync_copy(az_hbm, az_v)

    def compress(u, base, npts, xr, yr, zr, idx_ref):
        def bcast(r):
            return plsc.load_gather(
                ub_v, [jnp.full((L,), r * U + u, jnp.int32)])
        cx, cy, cz = bcast(0), bcast(1), bcast(2)
        hx, hy, hz = bcast(3) * 0.5, bcast(4) * 0.5, bcast(5) * 0.5
        lox, hix = cx - hx, cx + hx
        loy, hiy = cy - hy, cy + hy
        loz, hiz = cz - hz, cz + hz

        def one(j, cnt):
            xv = xr[pl.ds(j * L, L)]
            yv = yr[pl.ds(j * L, L)]
            zv = zr[pl.ds(j * L, L)]
            m = ((xv >= lox) & (xv <= hix) & (yv >= loy) & (yv <= hiy)
                 & (zv >= loz) & (zv <= hiz))
            ids = lax.iota(jnp.int32, L) + j * L
            plsc.store_compressed(idx_ref.at[pl.ds(base + cnt, L)], ids, mask=m)
            pc = plsc.all_reduce_population_count(m)
            return cnt + pc[0]

        def mk(jj, cnt):
            cnt = one(2 * jj, cnt)
            return one(2 * jj + 1, cnt)
        cnt = lax.fori_loop(0, npts // (2 * L), mk, jnp.int32(0))

        plsc.store_compressed(idx_ref.at[pl.ds(base + cnt, L)],
                              jnp.full((L,), npts, jnp.int32),
                              mask=cnt + lax.iota(jnp.int32, L) < npts)
        return cnt

    def pool(cnt, base, npts, idx_ref, tab_ref, nchan, out_ref, ob):
        nw = nchan // 2
        nvec = nw // L
        accs = tuple(jnp.full((2 * L,), I16MIN, jnp.int16)
                     for _ in range(nvec))

        def grp(t, accs):
            iv = idx_ref[pl.ds(base + t * L, L)]
            for lane in range(L):
                rb = iv[lane] * nw
                accs = tuple(
                    jnp.maximum(
                        accs[j],
                        plsc.bitcast(
                            tab_ref[pl.ds(
                                pl.multiple_of(rb + j * L, L), L)],
                            jnp.int16))
                    for j in range(nvec))
            return accs
        ngrp = (cnt + (L - 1)) >> 4
        accs = lax.fori_loop(0, ngrp, grp, accs)

        both_min = jnp.int32(-2147450880)
        fixw = jnp.full((L,), (cnt == npts).astype(jnp.int32) * both_min,
                        jnp.int32)
        fix = plsc.bitcast(fixw, jnp.int16)
        for j in range(nvec):
            out_ref[ob, pl.ds(j * L, L)] = plsc.bitcast(
                jnp.maximum(accs[j], fix), jnp.int32)

    cnts_s = [compress(ub_s + b, b * N, N, sx_v, sy_v, sz_v, idxs_v)
              for b in range(BPS)]
    cnts_a = [compress(ub_a + b, b * P, P, ax_v, ay_v, az_v, idxa_v)
              for b in range(BPW)]

    cp_a.wait()
    for b in range(BPW):
        pool(cnts_a[b], b * P, P, idxa_v, bfk_v, D, g2_v, b)
    cp_s.wait()
    for b in range(BPS):
        pool(cnts_s[b], b * N, N, idxs_v, sfk_v, CHH, g1_v, b)

    @pl.when(h == 0)
    def _():
        pltpu.sync_copy(g1_v, g1a_hbm.at[pl.ds(ub_s, BPS)])

    @pl.when(h != 0)
    def _():
        pltpu.sync_copy(g1_v, g1b_hbm.at[pl.ds(ub_s, BPS)])

    pltpu.sync_copy(g2_v, g2_hbm.at[pl.ds(ub_a, BPW)])


def _f16_key(x):
    u = lax.bitcast_convert_type(x, jnp.int32)
    s = lax.shift_right_logical(u, 31)
    e = lax.shift_right_logical(u, 23) & 0xFF
    m = u & 0x7FFFFF
    he = e - 112
    base = lax.shift_left(he, 10) | lax.shift_right_logical(m, 13)
    r = m & 0x1FFF
    lsb = lax.shift_right_logical(m, 13) & 1
    base = base + ((r > 0x1000) | ((r == 0x1000) & (lsb == 1))).astype(
        jnp.int32)
    base = jnp.where(he >= 31, 0x7C00, base)
    mf = m | 0x800000
    sh = jnp.clip(14 - he, 1, 31)
    subv = lax.shift_right_logical(mf, sh)
    subv = subv + (lax.shift_right_logical(mf, sh - 1) & 1)
    base = jnp.where(he <= 0, subv, base)
    return base ^ (0 - s)


def _key_to_f32(k):
    bits = jnp.where(k >= 0, k, 0x7FFF - k)
    s = lax.shift_right_logical(bits, 15) & 1
    e = lax.shift_right_logical(bits, 10) & 0x1F
    m = bits & 0x3FF
    fb = lax.shift_left(e + 112, 23) | lax.shift_left(m, 13)
    vn = lax.bitcast_convert_type(fb, jnp.float32)
    vs = m.astype(jnp.float32) * jnp.float32(2.0 ** -24)
    v = jnp.where(e == 0, vs, vn)
    return jnp.where(s == 1, -v, v)


def _pack_half(keys):
    lo = keys[:, 0:64] & 0xFFFF
    hi = lax.shift_left(keys[:, 64:128], 16)
    return lo | hi


_MINW = -2147450880


def _prep_body(sf_ref, bf_ref, sxyz_ref, axyz_ref, ub_ref,
               sfk0_ref, sfk1_ref, bfk_ref,
               sx_ref, sy_ref, sz_ref, ax_ref, ay_ref, az_ref, ub6_ref):
    skeys = _f16_key(sf_ref[:].T)
    sfk0_ref[0:N, :] = _pack_half(skeys[:, 0:CHH])
    sfk0_ref[N:NROW_S, :] = jnp.full((NROW_S - N, CHH // 2), _MINW, jnp.int32)
    sfk1_ref[0:N, :] = _pack_half(skeys[:, CHH:C])
    sfk1_ref[N:NROW_S, :] = jnp.full((NROW_S - N, CHH // 2), _MINW, jnp.int32)
    bkeys = _f16_key(bf_ref[:])
    bfk_ref[0:P, :] = _pack_half(bkeys)
    bfk_ref[P:NROW_A, :] = jnp.full((NROW_A - P, D // 2), _MINW, jnp.int32)
    st = sxyz_ref[:].T
    sx_ref[:], sy_ref[:], sz_ref[:] = st[0:1], st[1:2], st[2:3]
    at = axyz_ref[:].T
    ax_ref[:], ay_ref[:], az_ref[:] = at[0:1], at[1:2], at[2:3]
    ub6_ref[:] = ub_ref[:].T


def _finish_body(g1a_ref, g1b_ref, g2_ref, bfu_ref, w_ref, b_ref, out_ref):
    def unpack(w):
        lo = lax.shift_right_arithmetic(lax.shift_left(w, 16), 16)
        hi = lax.shift_right_arithmetic(w, 16)
        return _key_to_f32(lo), _key_to_f32(hi)

    a_lo, a_hi = unpack(g1a_ref[:])
    b_lo, b_hi = unpack(g1b_ref[:])
    c_lo, c_hi = unpack(g2_ref[:])
    glob = jnp.concatenate(
        [a_lo, a_hi, b_lo, b_hi, c_lo, c_hi, bfu_ref[:]], axis=1)
    out = lax.dot_general(glob, w_ref[:], (((1,), (1,)), ((), ())),
                          preferred_element_type=jnp.float32)
    out = out + b_ref[:]
    out_ref[:] = jax.nn.sigmoid(jnp.log(jnp.abs(out + 1e-6)))


def kernel(union_box, box_features, agg_xyz, seed_feature, seed_xyz,
           box_feature_union, W, b):
    f32, i32 = jnp.float32, jnp.int32
    sd = jax.ShapeDtypeStruct
    (sfk0, sfk1, bfk, sx, sy, sz, ax, ay, az, ub6) = pl.pallas_call(
        _prep_body,
        out_shape=(
            sd((NROW_S, CHH // 2), i32), sd((NROW_S, CHH // 2), i32),
            sd((NROW_A, D // 2), i32),
            sd((1, N), f32), sd((1, N), f32), sd((1, N), f32),
            sd((1, P), f32), sd((1, P), f32), sd((1, P), f32),
            sd((6, U), f32),
        ),
    )(seed_feature, box_features, seed_xyz, agg_xyz, union_box[0])

    g1a, g1b, g2k = _sc_pool(
        ub6.reshape(-1),
        sx.reshape(-1), sy.reshape(-1), sz.reshape(-1),
        ax.reshape(-1), ay.reshape(-1), az.reshape(-1),
        sfk0.reshape(-1), sfk1.reshape(-1), bfk.reshape(-1))

    return pl.pallas_call(
        _finish_body,
        out_shape=sd((U, OUTD), f32),
    )(g1a, g1b, g2k, box_feature_union[:, 0, :], W, b.reshape(1, OUTD))
